# speedup vs baseline: 1.3293x; 1.3293x over previous
import math

import jax
import jax.numpy as jnp
import numpy as np
from jax import lax
from jax.experimental import pallas as pl
from jax.experimental.pallas import tpu as pltpu
from jax.experimental.pallas import tpu_sc as plsc

_NUM_CORES = 2
_NUM_SUBCORES = 16
_NUM_WORKERS = _NUM_CORES * _NUM_SUBCORES
_LANES = 16

_MAX_LEN = 4096


def _pe_table(model_dim: int, max_len: int) -> np.ndarray:
    pos = np.arange(0, max_len, dtype=np.float32)[:, None]
    divterm = np.exp(
        np.arange(0, model_dim, 2, dtype=np.float32) * -(math.log(10000.0) / model_dim)
    )
    pe = np.zeros((max_len, model_dim), dtype=np.float32)
    pe[:, 0::2] = np.sin(pos * divterm)
    pe[:, 1::2] = np.cos(pos * divterm)
    return pe


_PE_NP = _pe_table(128, _MAX_LEN)


def _build_sc_call(n_tokens: int, seq: int, dim: int):
    bpw = n_tokens // _NUM_WORKERS
    chunks = bpw // 128
    mesh = plsc.VectorSubcoreMesh(core_axis_name="c", subcore_axis_name="s")

    def body(idx_hbm, table_hbm, pe_hbm, out_hbm, idx_v, acc_v, rows_v,
             sem_g, sem_p):
        wid = lax.axis_index("s") * _NUM_CORES + lax.axis_index("c")
        base = wid * bpw
        s0 = lax.rem(base, seq)

        pltpu.sync_copy(idx_hbm.at[pl.ds(wid * chunks, chunks), :], idx_v)

        copies = []
        for k in range(chunks):
            copies.append(
                pltpu.async_copy(
                    table_hbm.at[idx_v.at[k]],
                    rows_v.at[pl.ds(k * 128, 128), :],
                    sem_g,
                )
            )
        pltpu.async_copy(pe_hbm.at[pl.ds(s0, bpw), :], acc_v, sem_p).wait()
        for c in copies:
            c.wait()

        groups = dim // _LANES

        def add_row(i, carry):
            for j in range(groups):
                plsc.addupdate(
                    acc_v.at[i, pl.ds(j * _LANES, _LANES)],
                    rows_v[i, pl.ds(j * _LANES, _LANES)],
                )
            return carry

        lax.fori_loop(0, bpw, add_row, 0)

        pltpu.sync_copy(acc_v, out_hbm.at[pl.ds(base, bpw), :])

    call = pl.kernel(
        body,
        out_type=jax.ShapeDtypeStruct((n_tokens, dim), jnp.float32),
        mesh=mesh,
        scratch_types=[
            pltpu.VMEM((chunks, 128), jnp.int32),
            pltpu.VMEM((bpw, dim), jnp.float32),
            pltpu.VMEM((bpw, dim), jnp.float32),
            pltpu.SemaphoreType.DMA,
            pltpu.SemaphoreType.DMA,
        ],
    )
    return call


def kernel(tensor, table):
    batch, seq = tensor.shape
    vocab, dim = table.shape
    n_tokens = batch * seq
    idx = tensor.reshape(n_tokens // 128, 128).astype(jnp.int32)
    pe = jnp.asarray(_PE_NP[:seq, :dim])
    call = _build_sc_call(n_tokens, seq, dim)
    out = call(idx, table, pe)
    return out.reshape(batch, seq, dim)


# speedup vs baseline: 1.3691x; 1.0299x over previous
import math

import jax
import jax.numpy as jnp
import numpy as np
from jax import lax
from jax.experimental import pallas as pl
from jax.experimental.pallas import tpu as pltpu
from jax.experimental.pallas import tpu_sc as plsc

_NUM_CORES = 2
_NUM_SUBCORES = 16
_NUM_WORKERS = _NUM_CORES * _NUM_SUBCORES
_LANES = 16

_MAX_LEN = 4096
_CHUNK = 64
_ROW_UNROLL = 4


def _pe_table(model_dim: int, max_len: int) -> np.ndarray:
    pos = np.arange(0, max_len, dtype=np.float32)[:, None]
    divterm = np.exp(
        np.arange(0, model_dim, 2, dtype=np.float32) * -(math.log(10000.0) / model_dim)
    )
    pe = np.zeros((max_len, model_dim), dtype=np.float32)
    pe[:, 0::2] = np.sin(pos * divterm)
    pe[:, 1::2] = np.cos(pos * divterm)
    return pe


_PE_NP = _pe_table(128, _MAX_LEN)


def _build_sc_call(n_tokens: int, seq: int, dim: int):
    bpw = n_tokens // _NUM_WORKERS
    chunks = bpw // _CHUNK
    groups = dim // _LANES
    mesh = plsc.VectorSubcoreMesh(core_axis_name="c", subcore_axis_name="s")

    def body(idx_hbm, table_hbm, pe_hbm, out_hbm, idx_v, acc_v, rows_v,
             *sems):
        g_sems = sems[:chunks]
        p_sems = sems[chunks:2 * chunks]
        st_sem = sems[2 * chunks]
        wid = lax.axis_index("s") * _NUM_CORES + lax.axis_index("c")
        base = wid * bpw
        s0 = lax.rem(base, seq)

        pltpu.sync_copy(idx_hbm.at[pl.ds(wid * chunks, chunks), :], idx_v)

        g_cp = []
        p_cp = []
        for k in range(chunks):
            g_cp.append(
                pltpu.async_copy(
                    table_hbm.at[idx_v.at[k]],
                    rows_v.at[pl.ds(k * _CHUNK, _CHUNK), :],
                    g_sems[k],
                )
            )
            p_cp.append(
                pltpu.async_copy(
                    pe_hbm.at[pl.ds(s0 + k * _CHUNK, _CHUNK), :],
                    acc_v.at[pl.ds(k * _CHUNK, _CHUNK), :],
                    p_sems[k],
                )
            )

        st_cp = []
        for k in range(chunks):
            g_cp[k].wait()
            p_cp[k].wait()

            def add_rows(i, carry, k=k):
                r0 = k * _CHUNK + i * _ROW_UNROLL
                for u in range(_ROW_UNROLL):
                    for j in range(groups):
                        plsc.addupdate(
                            acc_v.at[r0 + u, pl.ds(j * _LANES, _LANES)],
                            rows_v[r0 + u, pl.ds(j * _LANES, _LANES)],
                        )
                return carry

            lax.fori_loop(0, _CHUNK // _ROW_UNROLL, add_rows, 0)
            st_cp.append(
                pltpu.async_copy(
                    acc_v.at[pl.ds(k * _CHUNK, _CHUNK), :],
                    out_hbm.at[pl.ds(base + k * _CHUNK, _CHUNK), :],
                    st_sem,
                )
            )
        for c in st_cp:
            c.wait()

    call = pl.kernel(
        body,
        out_type=jax.ShapeDtypeStruct((n_tokens, dim), jnp.float32),
        mesh=mesh,
        scratch_types=[
            pltpu.VMEM((chunks, _CHUNK), jnp.int32),
            pltpu.VMEM((bpw, dim), jnp.float32),
            pltpu.VMEM((bpw, dim), jnp.float32),
        ] + [pltpu.SemaphoreType.DMA] * (2 * chunks + 1),
    )
    return call


def kernel(tensor, table):
    batch, seq = tensor.shape
    vocab, dim = table.shape
    n_tokens = batch * seq
    idx = tensor.reshape(n_tokens // _CHUNK, _CHUNK).astype(jnp.int32)
    pe = jnp.asarray(_PE_NP[:seq, :dim])
    call = _build_sc_call(n_tokens, seq, dim)
    out = call(idx, table, pe)
    return out.reshape(batch, seq, dim)


# speedup vs baseline: 1.4990x; 1.0949x over previous
import math

import jax
import jax.numpy as jnp
import numpy as np
from jax import lax
from jax.experimental import pallas as pl
from jax.experimental.pallas import tpu as pltpu
from jax.experimental.pallas import tpu_sc as plsc

_NUM_CORES = 2
_NUM_SUBCORES = 16
_NUM_WORKERS = _NUM_CORES * _NUM_SUBCORES
_LANES = 16

_MAX_LEN = 4096


def _pe_table(model_dim: int, max_len: int) -> np.ndarray:
    pos = np.arange(0, max_len, dtype=np.float32)[:, None]
    divterm = np.exp(
        np.arange(0, model_dim, 2, dtype=np.float32) * -(math.log(10000.0) / model_dim)
    )
    pe = np.zeros((max_len, model_dim), dtype=np.float32)
    pe[:, 0::2] = np.sin(pos * divterm)
    pe[:, 1::2] = np.cos(pos * divterm)
    return pe


_PE_NP = _pe_table(128, _MAX_LEN)


def _build_sc_call(batch: int, seq: int, dim: int):
    ppw = seq // _NUM_WORKERS
    groups = dim // _LANES
    mesh = plsc.VectorSubcoreMesh(core_axis_name="c", subcore_axis_name="s")

    def body(idx_hbm, table_hbm, pe_hbm, out_hbm, idx_v, pe_v, rows_v,
             *sems):
        g_sems = sems[:batch]
        pe_sem = sems[batch]
        idx_sem = sems[batch + 1]
        st_sem = sems[batch + 2]
        wid = lax.axis_index("s") * _NUM_CORES + lax.axis_index("c")
        p0 = wid * ppw

        idx_cp = [
            pltpu.async_copy(idx_hbm.at[b, pl.ds(p0, ppw)], idx_v.at[b], idx_sem)
            for b in range(batch)
        ]
        pe_cp = pltpu.async_copy(pe_hbm.at[pl.ds(p0, ppw), :], pe_v, pe_sem)

        g_cp = []
        for b in range(batch):
            idx_cp[b].wait()
            g_cp.append(
                pltpu.async_copy(
                    table_hbm.at[idx_v.at[b]],
                    rows_v.at[pl.ds(b * ppw, ppw), :],
                    g_sems[b],
                )
            )
        pe_cp.wait()

        st_cp = []
        for b in range(batch):
            g_cp[b].wait()

            def add_rows(i, carry, b=b):
                for j in range(groups):
                    x = pe_v[i, pl.ds(j * _LANES, _LANES)]
                    plsc.addupdate(
                        rows_v.at[b * ppw + i, pl.ds(j * _LANES, _LANES)], x
                    )
                return carry

            lax.fori_loop(0, ppw, add_rows, 0)
            st_cp.append(
                pltpu.async_copy(
                    rows_v.at[pl.ds(b * ppw, ppw), :],
                    out_hbm.at[pl.ds(b * seq + p0, ppw), :],
                    st_sem,
                )
            )
        for c in st_cp:
            c.wait()

    call = pl.kernel(
        body,
        out_type=jax.ShapeDtypeStruct((batch * seq, dim), jnp.float32),
        mesh=mesh,
        scratch_types=[
            pltpu.VMEM((batch, ppw), jnp.int32),
            pltpu.VMEM((ppw, dim), jnp.float32),
            pltpu.VMEM((batch * ppw, dim), jnp.float32),
        ] + [pltpu.SemaphoreType.DMA] * (batch + 3),
    )
    return call


def kernel(tensor, table):
    batch, seq = tensor.shape
    vocab, dim = table.shape
    idx = tensor.astype(jnp.int32)
    pe = jnp.asarray(_PE_NP[:seq, :dim])
    call = _build_sc_call(batch, seq, dim)
    out = call(idx, table, pe)
    return out.reshape(batch, seq, dim)
